# baseline (speedup 1.0000x reference)
import jax
import jax.numpy as jnp
from jax import lax
from jax.experimental import pallas as pl
from jax.experimental.pallas import tpu as pltpu
from jax.experimental.pallas import tpu_sc as plsc

_CUT = 5
_E = 320000
_D = 128

_NC = 2
_NS = 16
_NW = _NC * _NS
_LANES = 16

_PER_W = _E // _NW
_BLK = 400
_NB = _PER_W // _BLK
_GPB = _BLK // _LANES


def _table_body(de_ref, w_ref, b_ref, t_ref):
    t_ref[...] = (
        jnp.dot(de_ref[...], w_ref[...], preferred_element_type=jnp.float32)
        + b_ref[...]
    )


def _lookup_body(t_hbm, feat_hbm, out_hbm, tbl_v, feat_v, rows_v, sem_f, sem_out):
    c = lax.axis_index("c")
    s = lax.axis_index("s")
    wid = c * _NS + s
    ebase = wid * _PER_W

    pltpu.sync_copy(t_hbm, tbl_v)
    pltpu.async_copy(feat_hbm.at[pl.ds(ebase, _BLK)], feat_v.at[pl.ds(0, _BLK)], sem_f)

    iota = lax.iota(jnp.int32, _LANES)
    iota128 = iota * 128

    def block(j, carry):
        b = j % 2
        foff = b * _BLK
        roff = b * _BLK * _D
        pltpu.make_async_copy(
            feat_hbm.at[pl.ds(ebase, _BLK)], feat_v.at[pl.ds(0, _BLK)], sem_f
        ).wait()

        @pl.when(j + 1 < _NB)
        def _():
            pltpu.async_copy(
                feat_hbm.at[pl.ds(ebase + (j + 1) * _BLK, _BLK)],
                feat_v.at[pl.ds((1 - b) * _BLK, _BLK)],
                sem_f,
            )

        @pl.when(j >= 2)
        def _():
            pltpu.make_async_copy(
                rows_v.at[pl.ds(roff, _BLK * _D)],
                out_hbm.at[pl.ds((ebase + (j - 2) * _BLK) * _D, _BLK * _D)],
                sem_out,
            ).wait()

        @plsc.parallel_loop(0, _GPB, 1, unroll=16)
        def _group(g):
            x = feat_v[pl.ds(foff + g * _LANES, _LANES)]
            xiv = jnp.clip(x, 1.0, _CUT - 1e-6).astype(jnp.int32) * _D
            ro0 = roff + g * (_LANES * _D)
            for l in range(_LANES):
                si = xiv[l]
                ro = ro0 + l * _D
                for c0 in range(0, _D, _LANES):
                    rows_v[pl.ds(ro + c0, _LANES)] = tbl_v[pl.ds(si + c0, _LANES)]
        pltpu.async_copy(
            rows_v.at[pl.ds(roff, _BLK * _D)],
            out_hbm.at[pl.ds((ebase + j * _BLK) * _D, _BLK * _D)],
            sem_out,
        )
        return carry

    lax.fori_loop(0, _NB, block, 0)
    for j in (_NB - 2, _NB - 1):
        roff = (j % 2) * _BLK * _D
        pltpu.make_async_copy(
            rows_v.at[pl.ds(roff, _BLK * _D)],
            out_hbm.at[pl.ds((ebase + j * _BLK) * _D, _BLK * _D)],
            sem_out,
        ).wait()


def kernel(dist_feat, dist_embed, W_in, b_in):
    table = pl.pallas_call(
        _table_body,
        out_shape=jax.ShapeDtypeStruct((_CUT + 1, _D), jnp.float32),
    )(dist_embed, W_in, b_in.reshape(1, _D))
    table = table.reshape((_CUT + 1) * _D)

    mesh = plsc.VectorSubcoreMesh(core_axis_name="c", subcore_axis_name="s")
    lookup = pl.kernel(
        _lookup_body,
        out_type=jax.ShapeDtypeStruct((_E * _D,), jnp.float32),
        mesh=mesh,
        compiler_params=pltpu.CompilerParams(needs_layout_passes=False),
        scratch_types=[
            pltpu.VMEM(((_CUT + 1) * _D,), jnp.float32),
            pltpu.VMEM((2 * _BLK,), jnp.float32),
            pltpu.VMEM((2 * _BLK * _D,), jnp.float32),
            pltpu.SemaphoreType.DMA,
            pltpu.SemaphoreType.DMA,
        ],
    )
    return lookup(table, dist_feat).reshape(_E, _D)


# speedup vs baseline: 3.5283x; 3.5283x over previous
import jax
import jax.numpy as jnp
from jax import lax
from jax.experimental import pallas as pl
from jax.experimental.pallas import tpu as pltpu
from jax.experimental.pallas import tpu_sc as plsc

_CUT = 5
_E = 320000
_D = 128

_NC = 2
_NS = 16
_NW = _NC * _NS
_LANES = 16

_PER_W = _E // _NW
_BLK = 400
_NB = _PER_W // _BLK
_GPB = _BLK // _LANES


def _table_body(de_ref, w_ref, b_ref, t_ref):
    t_ref[...] = (
        jnp.dot(de_ref[...], w_ref[...], preferred_element_type=jnp.float32)
        + b_ref[...]
    )


def _lookup_body(t_hbm, feat_hbm, out_hbm, tbl_v, feat_v, rows_v, sem_f, sem_out):
    c = lax.axis_index("c")
    s = lax.axis_index("s")
    wid = c * _NS + s
    ebase = wid * _PER_W

    pltpu.async_copy(feat_hbm.at[pl.ds(ebase, _BLK)], feat_v.at[pl.ds(0, _BLK)], sem_f)
    pltpu.sync_copy(t_hbm, tbl_v)

    def block(j, carry):
        b = j % 2
        foff = b * _BLK
        roff = b * _BLK * _D
        pltpu.make_async_copy(
            feat_hbm.at[pl.ds(ebase, _BLK)], feat_v.at[pl.ds(0, _BLK)], sem_f
        ).wait()

        @pl.when(j + 1 < _NB)
        def _():
            pltpu.async_copy(
                feat_hbm.at[pl.ds(ebase + (j + 1) * _BLK, _BLK)],
                feat_v.at[pl.ds((1 - b) * _BLK, _BLK)],
                sem_f,
            )

        @pl.when(j >= 2)
        def _():
            pltpu.make_async_copy(
                rows_v.at[pl.ds(roff, _BLK * _D)],
                out_hbm.at[pl.ds((ebase + (j - 2) * _BLK) * _D, _BLK * _D)],
                sem_out,
            ).wait()

        @plsc.parallel_loop(0, _GPB, 1)
        def _group(g):
            x = feat_v[pl.ds(foff + g * _LANES, _LANES)]
            xiv = jnp.clip(x, 1.0, _CUT - 1e-6).astype(jnp.int32) * _D
            ro0 = roff + g * (_LANES * _D)
            for l in range(_LANES):
                si = xiv[l]
                ro = ro0 + l * _D
                for c0 in range(0, _D, _LANES):
                    rows_v[pl.ds(ro + c0, _LANES)] = tbl_v[pl.ds(si + c0, _LANES)]
        pltpu.async_copy(
            rows_v.at[pl.ds(roff, _BLK * _D)],
            out_hbm.at[pl.ds((ebase + j * _BLK) * _D, _BLK * _D)],
            sem_out,
        )
        return carry

    lax.fori_loop(0, _NB, block, 0)
    for j in (_NB - 2, _NB - 1):
        roff = (j % 2) * _BLK * _D
        pltpu.make_async_copy(
            rows_v.at[pl.ds(roff, _BLK * _D)],
            out_hbm.at[pl.ds((ebase + j * _BLK) * _D, _BLK * _D)],
            sem_out,
        ).wait()


def kernel(dist_feat, dist_embed, W_in, b_in):
    table = pl.pallas_call(
        _table_body,
        out_shape=jax.ShapeDtypeStruct((_CUT + 1, _D), jnp.float32),
    )(dist_embed, W_in, b_in.reshape(1, _D))
    table = table.reshape((_CUT + 1) * _D)

    mesh = plsc.VectorSubcoreMesh(core_axis_name="c", subcore_axis_name="s")
    lookup = pl.kernel(
        _lookup_body,
        out_type=jax.ShapeDtypeStruct((_E * _D,), jnp.float32),
        mesh=mesh,
        compiler_params=pltpu.CompilerParams(needs_layout_passes=False),
        scratch_types=[
            pltpu.VMEM(((_CUT + 1) * _D,), jnp.float32),
            pltpu.VMEM((2 * _BLK,), jnp.float32),
            pltpu.VMEM((2 * _BLK * _D,), jnp.float32),
            pltpu.SemaphoreType.DMA,
            pltpu.SemaphoreType.DMA,
        ],
    )
    return lookup(table, dist_feat).reshape(_E, _D)
